# baseline (speedup 1.0000x reference)
import functools

import jax
import jax.numpy as jnp
from jax import lax
from jax.experimental import pallas as pl
from jax.experimental.pallas import tpu as pltpu
from jax.experimental.pallas import tpu_sc as plsc

_N = 10000
_E = 320000
_D = 128
_C = 16

_NC = 2
_NS = 16
_EPC = _E // _NC
_EPT = _EPC // _NS
_CH = 128
_NB = _EPT // _CH
_TE = _EPT - _NB * _CH
_DH = _D // 2
_EPT2 = _E // _NS
_NB2 = _EPT2 // _CH
_TE2 = _EPT2 - _NB2 * _CH
_RPT = 624
_TAIL0 = _NS * _RPT
_TAILN = _N - _TAIL0

_mesh = plsc.VectorSubcoreMesh(core_axis_name="c", subcore_axis_name="s")

_linear_params = pltpu.CompilerParams(use_tc_tiling_on_sc=False)


def _istart(src_hbm, dst_hbm, b, sb, db, sem):
  pltpu.make_async_copy(src_hbm.at[pl.ds(b, _CH)], sb, sem).start()
  pltpu.make_async_copy(dst_hbm.at[pl.ds(b, _CH)], db, sem).start()


def _iwait(src_hbm, dst_hbm, sb, db, sem):
  pltpu.make_async_copy(src_hbm.at[pl.ds(0, _CH)], sb, sem).wait()
  pltpu.make_async_copy(dst_hbm.at[pl.ds(0, _CH)], db, sem).wait()


def _deg_body(src_hbm, dst_hbm, zeros_hbm, ones_hbm, out_hbm,
              s0, d0, s1, d1, st, dt, ones_b, acc_s, acc_d, is0, is1):
  c = lax.axis_index("c")
  s = lax.axis_index("s")
  r0 = s * _RPT
  pltpu.sync_copy(zeros_hbm.at[pl.ds(r0, _RPT)], acc_s.at[pl.ds(r0, _RPT)])
  pltpu.sync_copy(zeros_hbm.at[pl.ds(r0, _RPT)], acc_d.at[pl.ds(r0, _RPT)])

  @pl.when(s == _NS - 1)
  def _():
    pltpu.sync_copy(zeros_hbm.at[pl.ds(_TAIL0, _TAILN)],
                    acc_s.at[pl.ds(_TAIL0, _TAILN)])
    pltpu.sync_copy(zeros_hbm.at[pl.ds(_TAIL0, _TAILN)],
                    acc_d.at[pl.ds(_TAIL0, _TAILN)])

  pltpu.sync_copy(ones_hbm, ones_b)
  plsc.subcore_barrier()

  base0 = c * _EPC + s * _EPT
  _istart(src_hbm, dst_hbm, base0, s0, d0, is0)
  _istart(src_hbm, dst_hbm, base0 + _CH, s1, d1, is1)

  @pl.loop(0, _NB - 2, step=2)
  def _(i):
    _iwait(src_hbm, dst_hbm, s0, d0, is0)
    pltpu.sync_copy(ones_b, acc_s.at[s0], add=True)
    pltpu.sync_copy(ones_b, acc_d.at[d0], add=True)
    _istart(src_hbm, dst_hbm, base0 + (i + 2) * _CH, s0, d0, is0)
    _iwait(src_hbm, dst_hbm, s1, d1, is1)
    pltpu.sync_copy(ones_b, acc_s.at[s1], add=True)
    pltpu.sync_copy(ones_b, acc_d.at[d1], add=True)
    _istart(src_hbm, dst_hbm, base0 + (i + 3) * _CH, s1, d1, is1)

  _iwait(src_hbm, dst_hbm, s0, d0, is0)
  pltpu.sync_copy(ones_b, acc_s.at[s0], add=True)
  pltpu.sync_copy(ones_b, acc_d.at[d0], add=True)
  _iwait(src_hbm, dst_hbm, s1, d1, is1)
  pltpu.sync_copy(ones_b, acc_s.at[s1], add=True)
  pltpu.sync_copy(ones_b, acc_d.at[d1], add=True)

  bt = base0 + _NB * _CH
  pltpu.sync_copy(src_hbm.at[pl.ds(bt, _TE)], st)
  pltpu.sync_copy(dst_hbm.at[pl.ds(bt, _TE)], dt)
  pltpu.sync_copy(ones_b.at[pl.ds(0, _TE)], acc_s.at[st], add=True)
  pltpu.sync_copy(ones_b.at[pl.ds(0, _TE)], acc_d.at[dt], add=True)

  plsc.subcore_barrier()
  pltpu.sync_copy(acc_s.at[pl.ds(r0, _RPT)], out_hbm.at[c, 0, pl.ds(r0, _RPT)])
  pltpu.sync_copy(acc_d.at[pl.ds(r0, _RPT)], out_hbm.at[c, 1, pl.ds(r0, _RPT)])

  @pl.when(s == _NS - 1)
  def _():
    pltpu.sync_copy(acc_s.at[pl.ds(_TAIL0, _TAILN)],
                    out_hbm.at[c, 0, pl.ds(_TAIL0, _TAILN)])
    pltpu.sync_copy(acc_d.at[pl.ds(_TAIL0, _TAILN)],
                    out_hbm.at[c, 1, pl.ds(_TAIL0, _TAILN)])


_deg_kernel = pl.kernel(
    _deg_body,
    out_type=jax.ShapeDtypeStruct((_NC, 2, _N, _C), jnp.float32),
    mesh=_mesh,
    scratch_types=[
        pltpu.VMEM((_CH,), jnp.int32),
        pltpu.VMEM((_CH,), jnp.int32),
        pltpu.VMEM((_CH,), jnp.int32),
        pltpu.VMEM((_CH,), jnp.int32),
        pltpu.VMEM((_TE,), jnp.int32),
        pltpu.VMEM((_TE,), jnp.int32),
        pltpu.VMEM((_CH, _C), jnp.float32),
        pltpu.VMEM_SHARED((_N, _C), jnp.float32),
        pltpu.VMEM_SHARED((_N, _C), jnp.float32),
        pltpu.SemaphoreType.DMA,
        pltpu.SemaphoreType.DMA,
    ],
    compiler_params=_linear_params,
)


def _segsum_pipelined(feat_ref, src_hbm, dst_hbm, base0,
                      s0, d0, s1, d1, st, dt, r0b, r1b, rt, acc,
                      is0, is1, gs0, gs1, nb=_NB, te=_TE):
  _istart(src_hbm, dst_hbm, base0, s0, d0, is0)
  _istart(src_hbm, dst_hbm, base0 + _CH, s1, d1, is1)
  _iwait(src_hbm, dst_hbm, s0, d0, is0)
  pltpu.make_async_copy(feat_ref.at[s0], r0b, gs0).start()

  @pl.loop(0, nb - 2, step=2)
  def _(i):
    _iwait(src_hbm, dst_hbm, s1, d1, is1)
    pltpu.make_async_copy(feat_ref.at[s0], r0b, gs0).wait()
    pltpu.make_async_copy(feat_ref.at[s1], r1b, gs1).start()
    pltpu.sync_copy(r0b, acc.at[d0], add=True)
    _istart(src_hbm, dst_hbm, base0 + (i + 2) * _CH, s0, d0, is0)
    _iwait(src_hbm, dst_hbm, s0, d0, is0)
    pltpu.make_async_copy(feat_ref.at[s1], r1b, gs1).wait()
    pltpu.make_async_copy(feat_ref.at[s0], r0b, gs0).start()
    pltpu.sync_copy(r1b, acc.at[d1], add=True)
    _istart(src_hbm, dst_hbm, base0 + (i + 3) * _CH, s1, d1, is1)

  _iwait(src_hbm, dst_hbm, s1, d1, is1)
  pltpu.make_async_copy(feat_ref.at[s0], r0b, gs0).wait()
  pltpu.make_async_copy(feat_ref.at[s1], r1b, gs1).start()
  pltpu.sync_copy(r0b, acc.at[d0], add=True)
  pltpu.make_async_copy(feat_ref.at[s1], r1b, gs1).wait()
  pltpu.sync_copy(r1b, acc.at[d1], add=True)

  bt = base0 + nb * _CH
  pltpu.sync_copy(src_hbm.at[pl.ds(bt, te)], st)
  pltpu.sync_copy(dst_hbm.at[pl.ds(bt, te)], dt)
  pltpu.async_copy(feat_ref.at[st], rt, gs0).wait()
  pltpu.sync_copy(rt, acc.at[dt], add=True)


def _segsum64_body(feat_hbm, src_hbm, dst_hbm, zeros_hbm, out_hbm,
                   s0, d0, s1, d1, st, dt, r0b, r1b, rt, xsh, acc,
                   is0, is1, gs0, gs1):
  c = lax.axis_index("c")
  s = lax.axis_index("s")
  r0 = s * _RPT
  pltpu.sync_copy(feat_hbm.at[c, pl.ds(r0, _RPT)], xsh.at[pl.ds(r0, _RPT)])
  pltpu.sync_copy(zeros_hbm.at[pl.ds(r0, _RPT)], acc.at[pl.ds(r0, _RPT)])

  @pl.when(s == _NS - 1)
  def _():
    pltpu.sync_copy(feat_hbm.at[c, pl.ds(_TAIL0, _TAILN)],
                    xsh.at[pl.ds(_TAIL0, _TAILN)])
    pltpu.sync_copy(zeros_hbm.at[pl.ds(_TAIL0, _TAILN)],
                    acc.at[pl.ds(_TAIL0, _TAILN)])

  plsc.subcore_barrier()
  base0 = s * _EPT2
  _segsum_pipelined(xsh, src_hbm, dst_hbm, base0,
                    s0, d0, s1, d1, st, dt, r0b, r1b, rt, acc,
                    is0, is1, gs0, gs1, nb=_NB2, te=_TE2)

  plsc.subcore_barrier()
  pltpu.sync_copy(acc.at[pl.ds(r0, _RPT)], out_hbm.at[c, pl.ds(r0, _RPT)])

  @pl.when(s == _NS - 1)
  def _():
    pltpu.sync_copy(acc.at[pl.ds(_TAIL0, _TAILN)],
                    out_hbm.at[c, pl.ds(_TAIL0, _TAILN)])


_segsum_d = pl.kernel(
    _segsum64_body,
    out_type=jax.ShapeDtypeStruct((_NC, _N, _DH), jnp.float32),
    mesh=_mesh,
    scratch_types=[
        pltpu.VMEM((_CH,), jnp.int32),
        pltpu.VMEM((_CH,), jnp.int32),
        pltpu.VMEM((_CH,), jnp.int32),
        pltpu.VMEM((_CH,), jnp.int32),
        pltpu.VMEM((_TE2,), jnp.int32),
        pltpu.VMEM((_TE2,), jnp.int32),
        pltpu.VMEM((_CH, _DH), jnp.float32),
        pltpu.VMEM((_CH, _DH), jnp.float32),
        pltpu.VMEM((_TE2, _DH), jnp.float32),
        pltpu.VMEM_SHARED((_N, _DH), jnp.float32),
        pltpu.VMEM_SHARED((_N, _DH), jnp.float32),
        pltpu.SemaphoreType.DMA,
        pltpu.SemaphoreType.DMA,
        pltpu.SemaphoreType.DMA,
        pltpu.SemaphoreType.DMA,
    ],
    compiler_params=_linear_params,
)


def _segsum16_body(z_hbm, src_hbm, dst_hbm, zeros_hbm, out_hbm,
                   s0, d0, s1, d1, st, dt, r0b, r1b, rt, zsh, acc,
                   is0, is1, gs0, gs1):
  c = lax.axis_index("c")
  s = lax.axis_index("s")
  r0 = s * _RPT
  pltpu.sync_copy(z_hbm.at[pl.ds(r0, _RPT)], zsh.at[pl.ds(r0, _RPT)])
  pltpu.sync_copy(zeros_hbm.at[pl.ds(r0, _RPT)], acc.at[pl.ds(r0, _RPT)])

  @pl.when(s == _NS - 1)
  def _():
    pltpu.sync_copy(z_hbm.at[pl.ds(_TAIL0, _TAILN)],
                    zsh.at[pl.ds(_TAIL0, _TAILN)])
    pltpu.sync_copy(zeros_hbm.at[pl.ds(_TAIL0, _TAILN)],
                    acc.at[pl.ds(_TAIL0, _TAILN)])

  plsc.subcore_barrier()
  base0 = c * _EPC + s * _EPT
  _segsum_pipelined(zsh, src_hbm, dst_hbm, base0,
                    s0, d0, s1, d1, st, dt, r0b, r1b, rt, acc,
                    is0, is1, gs0, gs1)

  plsc.subcore_barrier()
  pltpu.sync_copy(acc.at[pl.ds(r0, _RPT)], out_hbm.at[c, pl.ds(r0, _RPT)])

  @pl.when(s == _NS - 1)
  def _():
    pltpu.sync_copy(acc.at[pl.ds(_TAIL0, _TAILN)],
                    out_hbm.at[c, pl.ds(_TAIL0, _TAILN)])


_segsum_c = pl.kernel(
    _segsum16_body,
    out_type=jax.ShapeDtypeStruct((_NC, _N, _C), jnp.float32),
    mesh=_mesh,
    scratch_types=[
        pltpu.VMEM((_CH,), jnp.int32),
        pltpu.VMEM((_CH,), jnp.int32),
        pltpu.VMEM((_CH,), jnp.int32),
        pltpu.VMEM((_CH,), jnp.int32),
        pltpu.VMEM((_TE,), jnp.int32),
        pltpu.VMEM((_TE,), jnp.int32),
        pltpu.VMEM((_CH, _C), jnp.float32),
        pltpu.VMEM((_CH, _C), jnp.float32),
        pltpu.VMEM((_TE, _C), jnp.float32),
        pltpu.VMEM_SHARED((_N, _C), jnp.float32),
        pltpu.VMEM_SHARED((_N, _C), jnp.float32),
        pltpu.SemaphoreType.DMA,
        pltpu.SemaphoreType.DMA,
        pltpu.SemaphoreType.DMA,
        pltpu.SemaphoreType.DMA,
    ],
    compiler_params=_linear_params,
)


def _xw1_body(in_ref, w1, xw_ref):
  xw_ref[...] = jnp.dot(in_ref[...], w1[...],
                        preferred_element_type=jnp.float32)


_xw1 = pl.pallas_call(
    _xw1_body,
    out_shape=jax.ShapeDtypeStruct((_N, _D), jnp.float32),
)


def _prescale_body(xw_ref, deg_ref, xs_ref, ns_ref, nd_ref):
  deg = deg_ref[0] + deg_ref[1]
  ns = lax.rsqrt(jnp.maximum(deg[0, :, 0], 1.0)).reshape(_N, 1)
  nd = lax.rsqrt(jnp.maximum(deg[1, :, 0], 1.0)).reshape(_N, 1)
  ns_ref[...] = ns
  nd_ref[...] = nd
  scaled = xw_ref[...] * ns
  xs_ref[0] = scaled[:, :_DH]
  xs_ref[1] = scaled[:, _DH:]


_prescale = pl.pallas_call(
    _prescale_body,
    out_shape=[
        jax.ShapeDtypeStruct((2, _N, _DH), jnp.float32),
        jax.ShapeDtypeStruct((_N, 1), jnp.float32),
        jax.ShapeDtypeStruct((_N, 1), jnp.float32),
    ],
)


def _mlp_body(agg_ref, ns_ref, nd_ref, b1, w2, z_ref):
  agg = jnp.concatenate([agg_ref[0], agg_ref[1]], axis=1)
  h = jnp.maximum(agg * nd_ref[...] + b1[...], 0.0)
  z_ref[...] = jnp.dot(h, w2[...], preferred_element_type=jnp.float32) * ns_ref[...]


_mlp = pl.pallas_call(
    _mlp_body,
    out_shape=jax.ShapeDtypeStruct((_N, _C), jnp.float32),
)


def _readout_body(q_ref, nd_ref, b2, out_ref):
  h = jnp.maximum((q_ref[0] + q_ref[1]) * nd_ref[...] + b2[...], 0.0)
  out_ref[...] = jnp.sum(h, axis=0, keepdims=True)


_readout = pl.pallas_call(
    _readout_body,
    out_shape=jax.ShapeDtypeStruct((1, _C), jnp.float32),
)


@jax.jit
def kernel(in_feat, edge_index, W1, b1, W2, b2):
  src = edge_index[0]
  dst = edge_index[1]
  zeros16 = jnp.zeros((_N, 16), jnp.float32)
  zeros64 = jnp.zeros((_N, _DH), jnp.float32)
  ones_rows = jnp.ones((_CH, _C), jnp.float32)

  xw = _xw1(in_feat, W1)
  degpart = _deg_kernel(src, dst, zeros16, ones_rows)
  xs2, ns, nd = _prescale(xw, degpart)
  aggpart = _segsum_d(xs2, src, dst, zeros64)
  z = _mlp(aggpart, ns, nd, b1.reshape(1, _D), W2)
  qpart = _segsum_c(z, src, dst, zeros16)
  out = _readout(qpart, nd, b2.reshape(1, _C))
  return out.reshape(_C)


# speedup vs baseline: 1.2760x; 1.2760x over previous
import functools

import jax
import jax.numpy as jnp
from jax import lax
from jax.experimental import pallas as pl
from jax.experimental.pallas import tpu as pltpu
from jax.experimental.pallas import tpu_sc as plsc

_N = 10000
_E = 320000
_D = 128
_C = 16

_NC = 2
_NS = 16
_EPC = _E // _NC
_EPT = _EPC // _NS
_CH = 128
_NB = _EPT // _CH
_TE = _EPT - _NB * _CH
_RPT = 624
_TAIL0 = _NS * _RPT
_TAILN = _N - _TAIL0

_mesh = plsc.VectorSubcoreMesh(core_axis_name="c", subcore_axis_name="s")

_linear_params = pltpu.CompilerParams(use_tc_tiling_on_sc=False)


def _istart(src_hbm, dst_hbm, b, sb, db, sem):
  pltpu.make_async_copy(src_hbm.at[pl.ds(b, _CH)], sb, sem).start()
  pltpu.make_async_copy(dst_hbm.at[pl.ds(b, _CH)], db, sem).start()


def _iwait(src_hbm, dst_hbm, sb, db, sem):
  pltpu.make_async_copy(src_hbm.at[pl.ds(0, _CH)], sb, sem).wait()
  pltpu.make_async_copy(dst_hbm.at[pl.ds(0, _CH)], db, sem).wait()


def _deg_body(src_hbm, dst_hbm, zeros_hbm, ones_hbm, out_hbm,
              s0, d0, s1, d1, st, dt, ones_b, acc_s, acc_d, is0, is1):
  c = lax.axis_index("c")
  s = lax.axis_index("s")
  r0 = s * _RPT
  pltpu.sync_copy(zeros_hbm.at[pl.ds(r0, _RPT)], acc_s.at[pl.ds(r0, _RPT)])
  pltpu.sync_copy(zeros_hbm.at[pl.ds(r0, _RPT)], acc_d.at[pl.ds(r0, _RPT)])

  @pl.when(s == _NS - 1)
  def _():
    pltpu.sync_copy(zeros_hbm.at[pl.ds(_TAIL0, _TAILN)],
                    acc_s.at[pl.ds(_TAIL0, _TAILN)])
    pltpu.sync_copy(zeros_hbm.at[pl.ds(_TAIL0, _TAILN)],
                    acc_d.at[pl.ds(_TAIL0, _TAILN)])

  pltpu.sync_copy(ones_hbm, ones_b)
  plsc.subcore_barrier()

  base0 = c * _EPC + s * _EPT
  _istart(src_hbm, dst_hbm, base0, s0, d0, is0)
  _istart(src_hbm, dst_hbm, base0 + _CH, s1, d1, is1)

  @pl.loop(0, _NB - 2, step=2)
  def _(i):
    _iwait(src_hbm, dst_hbm, s0, d0, is0)
    pltpu.sync_copy(ones_b, acc_s.at[s0], add=True)
    pltpu.sync_copy(ones_b, acc_d.at[d0], add=True)
    _istart(src_hbm, dst_hbm, base0 + (i + 2) * _CH, s0, d0, is0)
    _iwait(src_hbm, dst_hbm, s1, d1, is1)
    pltpu.sync_copy(ones_b, acc_s.at[s1], add=True)
    pltpu.sync_copy(ones_b, acc_d.at[d1], add=True)
    _istart(src_hbm, dst_hbm, base0 + (i + 3) * _CH, s1, d1, is1)

  _iwait(src_hbm, dst_hbm, s0, d0, is0)
  pltpu.sync_copy(ones_b, acc_s.at[s0], add=True)
  pltpu.sync_copy(ones_b, acc_d.at[d0], add=True)
  _iwait(src_hbm, dst_hbm, s1, d1, is1)
  pltpu.sync_copy(ones_b, acc_s.at[s1], add=True)
  pltpu.sync_copy(ones_b, acc_d.at[d1], add=True)

  bt = base0 + _NB * _CH
  pltpu.sync_copy(src_hbm.at[pl.ds(bt, _TE)], st)
  pltpu.sync_copy(dst_hbm.at[pl.ds(bt, _TE)], dt)
  pltpu.sync_copy(ones_b.at[pl.ds(0, _TE)], acc_s.at[st], add=True)
  pltpu.sync_copy(ones_b.at[pl.ds(0, _TE)], acc_d.at[dt], add=True)

  plsc.subcore_barrier()
  pltpu.sync_copy(acc_s.at[pl.ds(r0, _RPT)], out_hbm.at[c, 0, pl.ds(r0, _RPT)])
  pltpu.sync_copy(acc_d.at[pl.ds(r0, _RPT)], out_hbm.at[c, 1, pl.ds(r0, _RPT)])

  @pl.when(s == _NS - 1)
  def _():
    pltpu.sync_copy(acc_s.at[pl.ds(_TAIL0, _TAILN)],
                    out_hbm.at[c, 0, pl.ds(_TAIL0, _TAILN)])
    pltpu.sync_copy(acc_d.at[pl.ds(_TAIL0, _TAILN)],
                    out_hbm.at[c, 1, pl.ds(_TAIL0, _TAILN)])


_deg_kernel = pl.kernel(
    _deg_body,
    out_type=jax.ShapeDtypeStruct((_NC, 2, _N, _C), jnp.float32),
    mesh=_mesh,
    scratch_types=[
        pltpu.VMEM((_CH,), jnp.int32),
        pltpu.VMEM((_CH,), jnp.int32),
        pltpu.VMEM((_CH,), jnp.int32),
        pltpu.VMEM((_CH,), jnp.int32),
        pltpu.VMEM((_TE,), jnp.int32),
        pltpu.VMEM((_TE,), jnp.int32),
        pltpu.VMEM((_CH, _C), jnp.float32),
        pltpu.VMEM_SHARED((_N, _C), jnp.float32),
        pltpu.VMEM_SHARED((_N, _C), jnp.float32),
        pltpu.SemaphoreType.DMA,
        pltpu.SemaphoreType.DMA,
    ],
    compiler_params=_linear_params,
)


def _segsum_pipelined(feat_ref, src_hbm, dst_hbm, base0,
                      s0, d0, s1, d1, st, dt, r0b, r1b, rt, acc,
                      is0, is1, gs0, gs1):
  _istart(src_hbm, dst_hbm, base0, s0, d0, is0)
  _istart(src_hbm, dst_hbm, base0 + _CH, s1, d1, is1)
  _iwait(src_hbm, dst_hbm, s0, d0, is0)
  pltpu.make_async_copy(feat_ref.at[s0], r0b, gs0).start()

  @pl.loop(0, _NB - 2, step=2)
  def _(i):
    _iwait(src_hbm, dst_hbm, s1, d1, is1)
    pltpu.make_async_copy(feat_ref.at[s0], r0b, gs0).wait()
    pltpu.make_async_copy(feat_ref.at[s1], r1b, gs1).start()
    pltpu.sync_copy(r0b, acc.at[d0], add=True)
    _istart(src_hbm, dst_hbm, base0 + (i + 2) * _CH, s0, d0, is0)
    _iwait(src_hbm, dst_hbm, s0, d0, is0)
    pltpu.make_async_copy(feat_ref.at[s1], r1b, gs1).wait()
    pltpu.make_async_copy(feat_ref.at[s0], r0b, gs0).start()
    pltpu.sync_copy(r1b, acc.at[d1], add=True)
    _istart(src_hbm, dst_hbm, base0 + (i + 3) * _CH, s1, d1, is1)

  _iwait(src_hbm, dst_hbm, s1, d1, is1)
  pltpu.make_async_copy(feat_ref.at[s0], r0b, gs0).wait()
  pltpu.make_async_copy(feat_ref.at[s1], r1b, gs1).start()
  pltpu.sync_copy(r0b, acc.at[d0], add=True)
  pltpu.make_async_copy(feat_ref.at[s1], r1b, gs1).wait()
  pltpu.sync_copy(r1b, acc.at[d1], add=True)

  bt = base0 + _NB * _CH
  pltpu.sync_copy(src_hbm.at[pl.ds(bt, _TE)], st)
  pltpu.sync_copy(dst_hbm.at[pl.ds(bt, _TE)], dt)
  pltpu.async_copy(feat_ref.at[st], rt, gs0).wait()
  pltpu.sync_copy(rt, acc.at[dt], add=True)


def _segsum_body(feat_hbm, src_hbm, dst_hbm, zeros_hbm, out_hbm,
                 s0, d0, s1, d1, st, dt, r0b, r1b, rt, acc,
                 is0, is1, gs0, gs1):
  c = lax.axis_index("c")
  s = lax.axis_index("s")
  r0 = s * _RPT
  pltpu.sync_copy(zeros_hbm.at[pl.ds(r0, _RPT)], acc.at[pl.ds(r0, _RPT)])

  @pl.when(s == _NS - 1)
  def _():
    pltpu.sync_copy(zeros_hbm.at[pl.ds(_TAIL0, _TAILN)],
                    acc.at[pl.ds(_TAIL0, _TAILN)])

  plsc.subcore_barrier()
  base0 = c * _EPC + s * _EPT
  _segsum_pipelined(feat_hbm, src_hbm, dst_hbm, base0,
                    s0, d0, s1, d1, st, dt, r0b, r1b, rt, acc,
                    is0, is1, gs0, gs1)

  plsc.subcore_barrier()
  pltpu.sync_copy(acc.at[pl.ds(r0, _RPT)], out_hbm.at[c, pl.ds(r0, _RPT)])

  @pl.when(s == _NS - 1)
  def _():
    pltpu.sync_copy(acc.at[pl.ds(_TAIL0, _TAILN)],
                    out_hbm.at[c, pl.ds(_TAIL0, _TAILN)])


_segsum_d = pl.kernel(
    _segsum_body,
    out_type=jax.ShapeDtypeStruct((_NC, _N, _D), jnp.float32),
    mesh=_mesh,
    scratch_types=[
        pltpu.VMEM((_CH,), jnp.int32),
        pltpu.VMEM((_CH,), jnp.int32),
        pltpu.VMEM((_CH,), jnp.int32),
        pltpu.VMEM((_CH,), jnp.int32),
        pltpu.VMEM((_TE,), jnp.int32),
        pltpu.VMEM((_TE,), jnp.int32),
        pltpu.VMEM((_CH, _D), jnp.float32),
        pltpu.VMEM((_CH, _D), jnp.float32),
        pltpu.VMEM((_TE, _D), jnp.float32),
        pltpu.VMEM_SHARED((_N, _D), jnp.float32),
        pltpu.SemaphoreType.DMA,
        pltpu.SemaphoreType.DMA,
        pltpu.SemaphoreType.DMA,
        pltpu.SemaphoreType.DMA,
    ],
)


def _segsum16_body(z_hbm, src_hbm, dst_hbm, zeros_hbm, out_hbm,
                   s0, d0, s1, d1, st, dt, r0b, r1b, rt, zsh, acc,
                   is0, is1, gs0, gs1):
  c = lax.axis_index("c")
  s = lax.axis_index("s")
  r0 = s * _RPT
  pltpu.sync_copy(z_hbm.at[pl.ds(r0, _RPT)], zsh.at[pl.ds(r0, _RPT)])
  pltpu.sync_copy(zeros_hbm.at[pl.ds(r0, _RPT)], acc.at[pl.ds(r0, _RPT)])

  @pl.when(s == _NS - 1)
  def _():
    pltpu.sync_copy(z_hbm.at[pl.ds(_TAIL0, _TAILN)],
                    zsh.at[pl.ds(_TAIL0, _TAILN)])
    pltpu.sync_copy(zeros_hbm.at[pl.ds(_TAIL0, _TAILN)],
                    acc.at[pl.ds(_TAIL0, _TAILN)])

  plsc.subcore_barrier()
  base0 = c * _EPC + s * _EPT
  _segsum_pipelined(zsh, src_hbm, dst_hbm, base0,
                    s0, d0, s1, d1, st, dt, r0b, r1b, rt, acc,
                    is0, is1, gs0, gs1)

  plsc.subcore_barrier()
  pltpu.sync_copy(acc.at[pl.ds(r0, _RPT)], out_hbm.at[c, pl.ds(r0, _RPT)])

  @pl.when(s == _NS - 1)
  def _():
    pltpu.sync_copy(acc.at[pl.ds(_TAIL0, _TAILN)],
                    out_hbm.at[c, pl.ds(_TAIL0, _TAILN)])


_segsum_c = pl.kernel(
    _segsum16_body,
    out_type=jax.ShapeDtypeStruct((_NC, _N, _C), jnp.float32),
    mesh=_mesh,
    scratch_types=[
        pltpu.VMEM((_CH,), jnp.int32),
        pltpu.VMEM((_CH,), jnp.int32),
        pltpu.VMEM((_CH,), jnp.int32),
        pltpu.VMEM((_CH,), jnp.int32),
        pltpu.VMEM((_TE,), jnp.int32),
        pltpu.VMEM((_TE,), jnp.int32),
        pltpu.VMEM((_CH, _C), jnp.float32),
        pltpu.VMEM((_CH, _C), jnp.float32),
        pltpu.VMEM((_TE, _C), jnp.float32),
        pltpu.VMEM_SHARED((_N, _C), jnp.float32),
        pltpu.VMEM_SHARED((_N, _C), jnp.float32),
        pltpu.SemaphoreType.DMA,
        pltpu.SemaphoreType.DMA,
        pltpu.SemaphoreType.DMA,
        pltpu.SemaphoreType.DMA,
    ],
    compiler_params=_linear_params,
)


def _xw1_body(in_ref, w1, xw_ref):
  xw_ref[...] = jnp.dot(in_ref[...], w1[...],
                        preferred_element_type=jnp.float32)


_xw1 = pl.pallas_call(
    _xw1_body,
    out_shape=jax.ShapeDtypeStruct((_N, _D), jnp.float32),
)


def _prescale_body(xw_ref, deg_ref, xs_ref, ns_ref, nd_ref):
  deg = deg_ref[0] + deg_ref[1]
  ns = lax.rsqrt(jnp.maximum(deg[0, :, 0], 1.0)).reshape(_N, 1)
  nd = lax.rsqrt(jnp.maximum(deg[1, :, 0], 1.0)).reshape(_N, 1)
  ns_ref[...] = ns
  nd_ref[...] = nd
  xs_ref[...] = xw_ref[...] * ns


_prescale = pl.pallas_call(
    _prescale_body,
    out_shape=[
        jax.ShapeDtypeStruct((_N, _D), jnp.float32),
        jax.ShapeDtypeStruct((_N, 1), jnp.float32),
        jax.ShapeDtypeStruct((_N, 1), jnp.float32),
    ],
)


def _mlp_body(agg_ref, ns_ref, nd_ref, b1, w2, z_ref):
  agg = agg_ref[0] + agg_ref[1]
  h = jnp.maximum(agg * nd_ref[...] + b1[...], 0.0)
  z_ref[...] = jnp.dot(h, w2[...], preferred_element_type=jnp.float32) * ns_ref[...]


_mlp = pl.pallas_call(
    _mlp_body,
    out_shape=jax.ShapeDtypeStruct((_N, _C), jnp.float32),
)


def _readout_body(q_ref, nd_ref, b2, out_ref):
  h = jnp.maximum((q_ref[0] + q_ref[1]) * nd_ref[...] + b2[...], 0.0)
  out_ref[...] = jnp.sum(h, axis=0, keepdims=True)


_readout = pl.pallas_call(
    _readout_body,
    out_shape=jax.ShapeDtypeStruct((1, _C), jnp.float32),
)


@jax.jit
def kernel(in_feat, edge_index, W1, b1, W2, b2):
  src = edge_index[0]
  dst = edge_index[1]
  zeros16 = jnp.zeros((_N, 16), jnp.float32)
  zeros128 = jnp.zeros((_N, _D), jnp.float32)
  ones_rows = jnp.ones((_CH, _C), jnp.float32)

  xw = _xw1(in_feat, W1)
  degpart = _deg_kernel(src, dst, zeros16, ones_rows)
  xs, ns, nd = _prescale(xw, degpart)
  aggpart = _segsum_d(xs, src, dst, zeros128)
  z = _mlp(aggpart, ns, nd, b1.reshape(1, _D), W2)
  qpart = _segsum_c(z, src, dst, zeros16)
  out = _readout(qpart, nd, b2.reshape(1, _C))
  return out.reshape(_C)
